# Initial kernel scaffold; baseline (speedup 1.0000x reference)
import jax
import jax.numpy as jnp
from jax.experimental import pallas as pl


def kernel(token_ids, E):
    raise NotImplementedError("write your pallas kernel here")


# speedup vs baseline: 1.0943x; 1.0943x over previous
import functools

import jax
import jax.numpy as jnp
from jax import lax
from jax.experimental import pallas as pl
from jax.experimental.pallas import tpu as pltpu
from jax.experimental.pallas import tpu_sc as plsc

NC, NS = 2, 16
NW = NC * NS
D = 32
IW = 128


@functools.cache
def _make_gather(B):
    assert B % (NW * IW) == 0
    b_per_w = B // NW
    C = 1024
    assert b_per_w % C == 0
    n_chunks = b_per_w // C
    G = C // IW

    mesh = plsc.VectorSubcoreMesh(core_axis_name="c", subcore_axis_name="s")

    @functools.partial(
        pl.kernel,
        out_type=jax.ShapeDtypeStruct((B, D), jnp.float32),
        mesh=mesh,
        scratch_types=[
            pltpu.VMEM((G, IW), jnp.int32),
            pltpu.VMEM((C, D), jnp.float32),
            pltpu.SemaphoreType.DMA,
        ],
        compiler_params=pltpu.CompilerParams(use_tc_tiling_on_sc=False),
    )
    def gather_kernel(idx_hbm, table_hbm, out_hbm, idx_v, rows_v, sem):
        wid = lax.axis_index("s") * NC + lax.axis_index("c")
        row_base = wid * b_per_w

        @pl.loop(0, n_chunks)
        def chunk(g):
            off = pl.multiple_of(row_base + g * C, C)
            pltpu.sync_copy(idx_hbm.at[pl.ds(pl.multiple_of(off // IW, 8), G), :], idx_v)
            copies = [
                pltpu.async_copy(
                    table_hbm.at[idx_v.at[j]],
                    rows_v.at[pl.ds(j * IW, IW), :],
                    sem,
                )
                for j in range(G)
            ]
            for cp in copies:
                cp.wait()
            pltpu.sync_copy(rows_v, out_hbm.at[pl.ds(off, C), :])

    return gather_kernel


def kernel(token_ids, E):
    B = token_ids.size
    idx = token_ids.reshape(B // IW, IW).astype(jnp.int32)
    out = _make_gather(B)(idx, E)
    return out.reshape(token_ids.shape + (D,))


# speedup vs baseline: 1.1106x; 1.0149x over previous
import functools

import jax
import jax.numpy as jnp
from jax import lax
from jax.experimental import pallas as pl
from jax.experimental.pallas import tpu as pltpu
from jax.experimental.pallas import tpu_sc as plsc

NC, NS = 2, 16
NW = NC * NS
D = 32
IW = 128


@functools.cache
def _make_gather(B):
    b_per_w = B // NW
    C = 512
    NBUF = 5
    G = C // IW
    n_chunks = b_per_w // C
    n_outer = n_chunks // NBUF
    idx_rows = b_per_w // IW
    assert B % (NW * IW) == 0 and n_outer * NBUF * C == b_per_w

    mesh = plsc.VectorSubcoreMesh(core_axis_name="c", subcore_axis_name="s")

    scratch = [pltpu.VMEM((idx_rows, IW), jnp.int32)]
    scratch += [pltpu.VMEM((C, D), jnp.float32) for _ in range(NBUF)]
    scratch += [pltpu.SemaphoreType.DMA for _ in range(2 * NBUF)]

    @functools.partial(
        pl.kernel,
        out_type=jax.ShapeDtypeStruct((B, D), jnp.float32),
        mesh=mesh,
        scratch_types=scratch,
        compiler_params=pltpu.CompilerParams(use_tc_tiling_on_sc=False),
    )
    def gather_kernel(idx_hbm, table_hbm, out_hbm, idx_all, *rest):
        rows = rest[:NBUF]
        sem_g = rest[NBUF:2 * NBUF]
        sem_s = rest[2 * NBUF:]
        wid = lax.axis_index("s") * NC + lax.axis_index("c")
        row_base = wid * b_per_w

        idx_base = pl.multiple_of(wid * idx_rows, 8)
        pltpu.sync_copy(idx_hbm.at[pl.ds(idx_base, idx_rows), :], idx_all)

        def fire_gathers(chunk, b):
            return [
                pltpu.async_copy(
                    table_hbm.at[idx_all.at[chunk * G + j]],
                    rows[b].at[pl.ds(j * IW, IW), :],
                    sem_g[b],
                )
                for j in range(G)
            ]

        @pl.loop(0, n_outer)
        def outer_loop(outer):
            descs = []
            for b in range(NBUF):
                @pl.when(outer != 0)
                def _wait_prev_store(b=b):
                    pltpu.make_async_copy(
                        rows[b], out_hbm.at[pl.ds(0, C), :], sem_s[b]
                    ).wait()

                descs.append(fire_gathers(outer * NBUF + b, b))
            for b in range(NBUF):
                for cp in descs[b]:
                    cp.wait()
                off = pl.multiple_of(row_base + (outer * NBUF + b) * C, C)
                pltpu.async_copy(rows[b], out_hbm.at[pl.ds(off, C), :], sem_s[b])

        for b in range(NBUF):
            pltpu.make_async_copy(
                rows[b], out_hbm.at[pl.ds(0, C), :], sem_s[b]
            ).wait()

    return gather_kernel


def kernel(token_ids, E):
    B = token_ids.size
    idx = token_ids.reshape(B // IW, IW).astype(jnp.int32)
    out = _make_gather(B)(idx, E)
    return out.reshape(token_ids.shape + (D,))


# speedup vs baseline: 1.8109x; 1.6305x over previous
import functools

import jax
import jax.numpy as jnp
from jax import lax
from jax.experimental import pallas as pl
from jax.experimental.pallas import tpu as pltpu
from jax.experimental.pallas import tpu_sc as plsc

NC, NS = 2, 16
NW = NC * NS
D = 32
IW = 128


@functools.cache
def _make_gather(B, V):
    b_per_w = B // NW
    C = 512
    NBUF = 5
    G = C // IW
    n_chunks = b_per_w // C
    n_outer = n_chunks // NBUF
    idx_rows = b_per_w // IW
    assert B % (NW * IW) == 0 and n_outer * NBUF * C == b_per_w

    mesh = plsc.VectorSubcoreMesh(core_axis_name="c", subcore_axis_name="s")

    scratch = [pltpu.VMEM((idx_rows, IW), jnp.int32)]
    scratch += [pltpu.VMEM((C, D), jnp.float32) for _ in range(NBUF)]
    scratch += [pltpu.SemaphoreType.DMA for _ in range(2 * NBUF)]

    @functools.partial(
        pl.kernel,
        out_type=jax.ShapeDtypeStruct((B, D), jnp.float32),
        mesh=mesh,
        scratch_types=scratch,
        compiler_params=pltpu.CompilerParams(use_tc_tiling_on_sc=False),
    )
    def gather_kernel(idx_hbm, table_hbm, out_hbm, idx_all, *rest):
        rows = rest[:NBUF]
        sem_g = rest[NBUF:2 * NBUF]
        sem_s = rest[2 * NBUF:]
        wid = lax.axis_index("s") * NC + lax.axis_index("c")
        row_base = wid * b_per_w

        idx_base = pl.multiple_of(wid * idx_rows, 8)
        pltpu.sync_copy(idx_hbm.at[pl.ds(idx_base, idx_rows), :], idx_all)

        def fire_gathers(chunk, b):
            return [
                pltpu.async_copy(
                    table_hbm.at[idx_all.at[chunk * G + j]],
                    rows[b].at[pl.ds(j * IW, IW), :],
                    sem_g[b],
                )
                for j in range(G)
            ]

        @pl.loop(0, n_outer)
        def outer_loop(outer):
            descs = []
            for b in range(NBUF):
                @pl.when(outer != 0)
                def _wait_prev_store(b=b):
                    pltpu.make_async_copy(
                        rows[b], out_hbm.at[pl.ds(0, C), :], sem_s[b]
                    ).wait()

                descs.append(fire_gathers(outer * NBUF + b, b))
            for b in range(NBUF):
                for cp in descs[b]:
                    cp.wait()
                off = pl.multiple_of(row_base + (outer * NBUF + b) * C, C)
                pltpu.async_copy(rows[b], out_hbm.at[pl.ds(off, C), :], sem_s[b])

        for b in range(NBUF):
            pltpu.make_async_copy(
                rows[b], out_hbm.at[pl.ds(0, C), :], sem_s[b]
            ).wait()

    return gather_kernel


def kernel(token_ids, E):
    V = E.shape[0]
    B = token_ids.size
    table_wide = lax.optimization_barrier(E.reshape(V * D // 128, 128))
    table = table_wide.reshape(V, D)
    idx = token_ids.reshape(B // IW, IW).astype(jnp.int32)
    out = _make_gather(B, V)(idx, table)
    out_wide = lax.optimization_barrier(out.reshape(B * D // 128, 128))
    return out_wide.reshape(token_ids.shape + (D,))
